# Initial kernel scaffold; baseline (speedup 1.0000x reference)
import jax
import jax.numpy as jnp
from jax.experimental import pallas as pl


def kernel(ag_type, trajs, scores):
    raise NotImplementedError("write your pallas kernel here")


# speedup vs baseline: 1.4150x; 1.4150x over previous
import functools

import jax
import jax.numpy as jnp
from jax import lax
from jax.experimental import pallas as pl
from jax.experimental.pallas import tpu as pltpu
from jax.experimental.pallas import tpu_sc as plsc

_S, _K, _A, _T, _C = 32, 64, 64, 80, 3
_KP = 6
_NMS_THRESH = (2.5, 1.0, 2.0)
_ROW = _T * _C
_KEEP = 16 * _C
_B = _S * _A * _KP
_NW = 32
_CHUNK = 128
_NCH = _B // (_NW * _CHUNK)


def _nms_body(sc_ref, xs_ref, ys_ref, agt_ref, fidx_ref, sout_ref):
    s = pl.program_id(0)
    sc_raw = sc_ref[0]
    xs = xs_ref[0]
    ys = ys_ref[0]
    agt = agt_ref[0]
    thresh = (_NMS_THRESH[0] * agt[0:1, :]
              + _NMS_THRESH[1] * agt[1:2, :]
              + _NMS_THRESH[2] * agt[2:3, :])

    m = jnp.max(sc_raw, axis=0, keepdims=True)
    e = jnp.exp(sc_raw - m)
    p = e / jnp.sum(e, axis=0, keepdims=True)

    kiota = lax.broadcasted_iota(jnp.int32, (_K, _A), 0)
    aiota = lax.broadcasted_iota(jnp.int32, (1, _A), 1)

    scn = p
    psel = []
    for j in range(_KP):
        mx = jnp.max(scn, axis=0, keepdims=True)
        idx = jnp.min(jnp.where(scn == mx, kiota, _K), axis=0, keepdims=True)
        oh = kiota == idx
        xsel = jnp.sum(jnp.where(oh, xs, 0.0), axis=0, keepdims=True)
        ysel = jnp.sum(jnp.where(oh, ys, 0.0), axis=0, keepdims=True)
        psel.append(jnp.sum(jnp.where(oh, p, 0.0), axis=0, keepdims=True))
        dx = xs - xsel
        dy = ys - ysel
        drow = jnp.sqrt(dx * dx + dy * dy)
        within = drow < thresh
        scn = scn * jnp.where(within, 0.01, 1.0)
        scn = jnp.where(oh, -1.0, scn)
        fidx_ref[0, j:j + 1, :] = s * (_K * _A) + idx * _A + aiota

    pm = psel[0]
    for j in range(1, _KP):
        pm = jnp.maximum(pm, psel[j])
    r2 = [(pj / pm) * (pj / pm) for pj in psel]
    tot = r2[0]
    for j in range(1, _KP):
        tot = tot + r2[j]
    for j in range(_KP):
        sout_ref[0, j:j + 1, :] = r2[j] / tot


_nms_call = pl.pallas_call(
    _nms_body,
    grid=(_S,),
    in_specs=[
        pl.BlockSpec((1, _K, _A), lambda s: (s, 0, 0)),
        pl.BlockSpec((1, _K, _A), lambda s: (s, 0, 0)),
        pl.BlockSpec((1, _K, _A), lambda s: (s, 0, 0)),
        pl.BlockSpec((1, _C, _A), lambda s: (s, 0, 0)),
    ],
    out_specs=[
        pl.BlockSpec((1, _KP, _A), lambda s: (s, 0, 0)),
        pl.BlockSpec((1, _KP, _A), lambda s: (s, 0, 0)),
    ],
    out_shape=[
        jax.ShapeDtypeStruct((_S, _KP, _A), jnp.int32),
        jax.ShapeDtypeStruct((_S, _KP, _A), jnp.float32),
    ],
)


def _sc_gather_body(table_hbm, idx_hbm, out_hbm, idx_v, rows_v, out_v, sem):
    wid = lax.axis_index("s") * 2 + lax.axis_index("c")
    pltpu.sync_copy(idx_hbm.at[wid], idx_v)
    copies = [
        pltpu.async_copy(table_hbm.at[idx_v.at[j]], rows_v.at[j], sem)
        for j in range(_NCH)
    ]
    for cp in copies:
        cp.wait()

    lane = lax.iota(jnp.int32, 16)
    srcs = []
    for v in range(_C):
        pos = lane + v * 16
        c3 = pos % 3
        t5 = (pos - c3) // 3
        srcs.append(12 + 15 * t5 + c3)

    for j in range(_NCH):
        jf = jnp.full((16,), j, jnp.int32)

        def body(rl, carry, jf=jf):
            rf = jnp.full((16,), rl, jnp.int32)
            for v in range(_C):
                out_v[j, rl, pl.ds(v * 16, 16)] = plsc.load_gather(
                    rows_v, [jf, rf, srcs[v]])
            return carry

        lax.fori_loop(0, _CHUNK, body, 0)

    pltpu.sync_copy(out_v, out_hbm.at[wid])


@functools.cache
def _sc_gather():
    return functools.partial(
        pl.kernel,
        mesh=plsc.VectorSubcoreMesh(core_axis_name="c", subcore_axis_name="s"),
        out_type=jax.ShapeDtypeStruct((_NW, _NCH, _CHUNK, _KEEP), jnp.float32),
        compiler_params=pltpu.CompilerParams(use_tc_tiling_on_sc=False, needs_layout_passes=False),
        scratch_types=[
            pltpu.VMEM((_NCH, _CHUNK), jnp.int32),
            pltpu.VMEM((_NCH, _CHUNK, _ROW), jnp.float32),
            pltpu.VMEM((_NCH, _CHUNK, _KEEP), jnp.float32),
            pltpu.SemaphoreType.DMA,
        ],
    )(_sc_gather_body)


def kernel(ag_type, trajs, scores):
    xs = trajs[:, :, :, _T - 1, 0]
    ys = trajs[:, :, :, _T - 1, 1]
    agt = jnp.swapaxes(ag_type, 1, 2)
    fidx, sout = _nms_call(scores, xs, ys, agt)
    scores_k = jnp.swapaxes(sout, 1, 2)
    flat_idx = jnp.transpose(fidx, (0, 2, 1)).reshape(_NW, _NCH, _CHUNK)
    table = trajs.reshape(_S * _K * _A, _ROW)
    rows = _sc_gather()(table, flat_idx)
    trajs_out = rows.reshape(_S, _A, _KP, 16, _C)
    return trajs_out, scores_k
